# Initial kernel scaffold; baseline (speedup 1.0000x reference)
import functools

import jax
import jax.numpy as jnp
from jax import lax
from jax.experimental import pallas as pl
from jax.experimental.pallas import tpu as pltpu
from jax.experimental.pallas import tpu_sc as plsc

N_NODES = 10000
N_EDGES = 320000
FEAT = 128
NC = 2
NS = 16
NW = NC * NS
EPW = N_EDGES // NW
K = 80
NCH = EPW // K
RPT = N_NODES // NS
DEGW = 16


def _sc_mesh():
    return plsc.VectorSubcoreMesh(
        core_axis_name="c", subcore_axis_name="s", num_cores=NC, num_subcores=NS
    )


@functools.partial(
    pl.kernel,
    out_type=jax.ShapeDtypeStruct((NC, NS, RPT, DEGW), jnp.float32),
    mesh=_sc_mesh(),
    scratch_types=[
        pltpu.VMEM_SHARED((N_NODES, DEGW), jnp.float32),
        pltpu.VMEM((1, K), jnp.int32),
        pltpu.VMEM((K, DEGW), jnp.float32),
    ],
)
def _deg_kernel(dst_hbm, zero_hbm, out_hbm, acc, idx_d, ones):
    c = lax.axis_index("c")
    s = lax.axis_index("s")
    wid = c * NS + s

    def fill(i, carry):
        ones[i] = jnp.ones((DEGW,), jnp.float32)
        return carry

    lax.fori_loop(0, K, fill, 0)
    pltpu.sync_copy(zero_hbm, acc.at[pl.ds(s * RPT, RPT)])
    plsc.subcore_barrier()

    base = wid * EPW

    def body(i, carry):
        off = base + i * K
        pltpu.sync_copy(dst_hbm.at[pl.ds(off, K)], idx_d.at[0])
        pltpu.sync_copy(ones, acc.at[idx_d.at[0]], add=True)
        return carry

    lax.fori_loop(0, NCH, body, 0)
    plsc.subcore_barrier()
    pltpu.sync_copy(acc.at[pl.ds(s * RPT, RPT)], out_hbm.at[c, s])


@functools.partial(
    pl.kernel,
    out_type=jax.ShapeDtypeStruct((NC, NS, RPT, FEAT), jnp.float32),
    mesh=_sc_mesh(),
    scratch_types=[
        pltpu.VMEM_SHARED((N_NODES, FEAT), jnp.float32),
        pltpu.VMEM((1, K), jnp.int32),
        pltpu.VMEM((1, K), jnp.int32),
        pltpu.VMEM((1, K, FEAT), jnp.float32),
        pltpu.SemaphoreType.DMA,
    ],
)
def _agg_kernel(g_hbm, src_hbm, dst_hbm, zero_hbm, out_hbm, acc, idx_s, idx_d, rows, sem):
    c = lax.axis_index("c")
    s = lax.axis_index("s")
    wid = c * NS + s

    pltpu.sync_copy(zero_hbm, acc.at[pl.ds(s * RPT, RPT)])
    plsc.subcore_barrier()

    base = wid * EPW

    def body(i, carry):
        off = base + i * K
        pltpu.sync_copy(src_hbm.at[pl.ds(off, K)], idx_s.at[0])
        pltpu.async_copy(g_hbm.at[idx_s.at[0]], rows.at[0], sem).wait()
        pltpu.sync_copy(dst_hbm.at[pl.ds(off, K)], idx_d.at[0])
        pltpu.sync_copy(rows.at[0], acc.at[idx_d.at[0]], add=True)
        return carry

    lax.fori_loop(0, NCH, body, 0)
    plsc.subcore_barrier()
    pltpu.sync_copy(acc.at[pl.ds(s * RPT, RPT)], out_hbm.at[c, s])


BM = 400


def _dinv_from(degp):
    deg = degp[0, :, 0] + degp[1, :, 0] + 1.0
    return lax.rsqrt(deg)


def _tc_a_body(x_ref, w_ref, degp_ref, o_ref):
    dinv = _dinv_from(degp_ref[...])
    h = jnp.dot(x_ref[...], w_ref[...], preferred_element_type=jnp.float32)
    o_ref[...] = h * dinv[:, None]


def _tc_b_body(p_ref, g_ref, degp_ref, b_ref, w_ref, o_ref):
    dinv = _dinv_from(degp_ref[...])
    p = p_ref[...]
    t = (p[0] + p[1] + g_ref[...]) * dinv[:, None] + b_ref[...]
    t = jnp.maximum(t, 0.0)
    o_ref[...] = jnp.dot(t, w_ref[...], preferred_element_type=jnp.float32) * dinv[:, None]


def _tc_c_body(p_ref, g_ref, degp_ref, b_ref, w_ref, bfc_ref, o_ref):
    dinv = _dinv_from(degp_ref[...])
    p = p_ref[...]
    t = (p[0] + p[1] + g_ref[...]) * dinv[:, None] + b_ref[...]
    t = jnp.maximum(t, 0.0)
    o_ref[...] = jnp.dot(t, w_ref[...], preferred_element_type=jnp.float32) + bfc_ref[...]


_BS_ROW = pl.BlockSpec((BM, FEAT), lambda i: (i, 0))
_BS_W = pl.BlockSpec((FEAT, FEAT), lambda i: (0, 0))
_BS_DEGP = pl.BlockSpec((NC, BM, DEGW), lambda i: (0, i, 0))
_BS_PART = pl.BlockSpec((NC, BM, FEAT), lambda i: (0, i, 0))
_BS_BIAS = pl.BlockSpec((1, FEAT), lambda i: (0, 0))
_OUT_SHAPE = jax.ShapeDtypeStruct((N_NODES, FEAT), jnp.float32)


def _tc_a(x, w1, degp):
    return pl.pallas_call(
        _tc_a_body,
        grid=(N_NODES // BM,),
        in_specs=[_BS_ROW, _BS_W, _BS_DEGP],
        out_specs=_BS_ROW,
        out_shape=_OUT_SHAPE,
    )(x, w1, degp)


def _tc_b(p, g, degp, b, w):
    return pl.pallas_call(
        _tc_b_body,
        grid=(N_NODES // BM,),
        in_specs=[_BS_PART, _BS_ROW, _BS_DEGP, _BS_BIAS, _BS_W],
        out_specs=_BS_ROW,
        out_shape=_OUT_SHAPE,
    )(p, g, degp, b, w)


def _tc_c(p, g, degp, b, w, bfc):
    return pl.pallas_call(
        _tc_c_body,
        grid=(N_NODES // BM,),
        in_specs=[_BS_PART, _BS_ROW, _BS_DEGP, _BS_BIAS, _BS_W, _BS_BIAS],
        out_specs=_BS_ROW,
        out_shape=_OUT_SHAPE,
    )(p, g, degp, b, w, bfc)


def kernel(x, edge_index, W1, b1, W2, b2, Wfc, bfc):
    src = edge_index[0]
    dst = edge_index[1]
    zero_f = jnp.zeros((RPT, FEAT), jnp.float32)
    zero_d = jnp.zeros((RPT, DEGW), jnp.float32)

    degp = _deg_kernel(dst, zero_d).reshape(NC, N_NODES, DEGW)
    g1 = _tc_a(x, W1, degp)
    p1 = _agg_kernel(g1, src, dst, zero_f).reshape(NC, N_NODES, FEAT)
    g2 = _tc_b(p1, g1, degp, b1.reshape(1, FEAT), W2)
    p2 = _agg_kernel(g2, src, dst, zero_f).reshape(NC, N_NODES, FEAT)
    out = _tc_c(p2, g2, degp, b2.reshape(1, FEAT), Wfc, bfc.reshape(1, FEAT))
    return out


# speedup vs baseline: 12.0509x; 12.0509x over previous
import functools

import jax
import jax.numpy as jnp
from jax import lax
from jax.experimental import pallas as pl
from jax.experimental.pallas import tpu as pltpu
from jax.experimental.pallas import tpu_sc as plsc

N_NODES = 10000
N_EDGES = 320000
FEAT = 128
NC = 2
NS = 16
NW = NC * NS
EPW = N_EDGES // NW
K = 80
NCH = EPW // K
RPT = N_NODES // NS
DEGW = 16


def _sc_mesh():
    return plsc.VectorSubcoreMesh(
        core_axis_name="c", subcore_axis_name="s", num_cores=NC, num_subcores=NS
    )


def _deg_body(dst_hbm, zero_hbm, ones_hbm, out_hbm, acc, idx_d, ones):
    c = lax.axis_index("c")
    s = lax.axis_index("s")
    wid = c * NS + s

    pltpu.sync_copy(ones_hbm, ones)
    pltpu.sync_copy(zero_hbm, acc.at[pl.ds(s * RPT, RPT)])
    plsc.subcore_barrier()

    base = wid * EPW

    def body(i, carry):
        off = base + i * K
        pltpu.sync_copy(dst_hbm.at[pl.ds(off, K)], idx_d.at[0])
        pltpu.sync_copy(ones, acc.at[idx_d.at[0]], add=True)
        return carry

    lax.fori_loop(0, NCH, body, 0)
    plsc.subcore_barrier()
    pltpu.sync_copy(acc.at[pl.ds(s * RPT, RPT)], out_hbm.at[c, s])


def _agg_body(g_hbm, src_hbm, dst_hbm, zero_hbm, out_hbm, acc, idx_s, idx_d, rows, sem):
    c = lax.axis_index("c")
    s = lax.axis_index("s")
    wid = c * NS + s

    pltpu.sync_copy(zero_hbm, acc.at[pl.ds(s * RPT, RPT)])
    plsc.subcore_barrier()

    base = wid * EPW

    def body(i, carry):
        off = base + i * K
        pltpu.sync_copy(src_hbm.at[pl.ds(off, K)], idx_s.at[0])
        pltpu.async_copy(g_hbm.at[idx_s.at[0]], rows.at[0], sem).wait()
        pltpu.sync_copy(dst_hbm.at[pl.ds(off, K)], idx_d.at[0])
        pltpu.sync_copy(rows.at[0], acc.at[idx_d.at[0]], add=True)
        return carry

    lax.fori_loop(0, NCH, body, 0)
    plsc.subcore_barrier()
    pltpu.sync_copy(acc.at[pl.ds(s * RPT, RPT)], out_hbm.at[c, s])


def _make_deg_kernel(interpret=False):
    return functools.partial(
        pl.kernel,
        out_type=jax.ShapeDtypeStruct((NC, NS, RPT, DEGW), jnp.float32),
        mesh=_sc_mesh(),
        scratch_types=[
            pltpu.VMEM_SHARED((N_NODES, DEGW), jnp.float32),
            pltpu.VMEM((1, K), jnp.int32),
            pltpu.VMEM((K, DEGW), jnp.float32),
        ],
        interpret=interpret,
    )(_deg_body)


def _make_agg_kernel(interpret=False):
    return functools.partial(
        pl.kernel,
        out_type=jax.ShapeDtypeStruct((NC, NS, RPT, FEAT), jnp.float32),
        mesh=_sc_mesh(),
        scratch_types=[
            pltpu.VMEM_SHARED((N_NODES, FEAT), jnp.float32),
            pltpu.VMEM((1, K), jnp.int32),
            pltpu.VMEM((1, K), jnp.int32),
            pltpu.VMEM((1, K, FEAT), jnp.float32),
            pltpu.SemaphoreType.DMA,
        ],
        interpret=interpret,
    )(_agg_body)


_deg_kernel = _make_deg_kernel()
_agg_kernel = _make_agg_kernel()


BM = 400


def _dinv_from(degp):
    deg = degp[0, :, 0] + degp[1, :, 0] + 1.0
    return lax.rsqrt(deg)


def _tc_a_body(x_ref, w_ref, degp_ref, o_ref):
    dinv = _dinv_from(degp_ref[...])
    h = jnp.dot(x_ref[...], w_ref[...], preferred_element_type=jnp.float32)
    o_ref[...] = h * dinv[:, None]


def _tc_b_body(p_ref, g_ref, degp_ref, b_ref, w_ref, o_ref):
    dinv = _dinv_from(degp_ref[...])
    p = p_ref[...]
    t = (p[0] + p[1] + g_ref[...]) * dinv[:, None] + b_ref[...]
    t = jnp.maximum(t, 0.0)
    o_ref[...] = jnp.dot(t, w_ref[...], preferred_element_type=jnp.float32) * dinv[:, None]


def _tc_c_body(p_ref, g_ref, degp_ref, b_ref, w_ref, bfc_ref, o_ref):
    dinv = _dinv_from(degp_ref[...])
    p = p_ref[...]
    t = (p[0] + p[1] + g_ref[...]) * dinv[:, None] + b_ref[...]
    t = jnp.maximum(t, 0.0)
    o_ref[...] = jnp.dot(t, w_ref[...], preferred_element_type=jnp.float32) + bfc_ref[...]


_BS_ROW = pl.BlockSpec((BM, FEAT), lambda i: (i, 0))
_BS_W = pl.BlockSpec((FEAT, FEAT), lambda i: (0, 0))
_BS_DEGP = pl.BlockSpec((NC, BM, DEGW), lambda i: (0, i, 0))
_BS_PART = pl.BlockSpec((NC, BM, FEAT), lambda i: (0, i, 0))
_BS_BIAS = pl.BlockSpec((1, FEAT), lambda i: (0, 0))
_OUT_SHAPE = jax.ShapeDtypeStruct((N_NODES, FEAT), jnp.float32)


def _tc_a(x, w1, degp):
    return pl.pallas_call(
        _tc_a_body,
        grid=(N_NODES // BM,),
        in_specs=[_BS_ROW, _BS_W, _BS_DEGP],
        out_specs=_BS_ROW,
        out_shape=_OUT_SHAPE,
    )(x, w1, degp)


def _tc_b(p, g, degp, b, w):
    return pl.pallas_call(
        _tc_b_body,
        grid=(N_NODES // BM,),
        in_specs=[_BS_PART, _BS_ROW, _BS_DEGP, _BS_BIAS, _BS_W],
        out_specs=_BS_ROW,
        out_shape=_OUT_SHAPE,
    )(p, g, degp, b, w)


def _tc_c(p, g, degp, b, w, bfc):
    return pl.pallas_call(
        _tc_c_body,
        grid=(N_NODES // BM,),
        in_specs=[_BS_PART, _BS_ROW, _BS_DEGP, _BS_BIAS, _BS_W, _BS_BIAS],
        out_specs=_BS_ROW,
        out_shape=_OUT_SHAPE,
    )(p, g, degp, b, w, bfc)


def kernel(x, edge_index, W1, b1, W2, b2, Wfc, bfc):
    src = edge_index[0]
    dst = edge_index[1]
    zero_f = jnp.zeros((RPT, FEAT), jnp.float32)
    zero_d = jnp.zeros((RPT, DEGW), jnp.float32)

    degp = _deg_kernel(
        dst, zero_d, jnp.ones((K, DEGW), jnp.float32)
    ).reshape(NC, N_NODES, DEGW)

    def _agg(g):
        return _agg_kernel(g, src, dst, zero_f).reshape(NC, N_NODES, FEAT)
    g1 = _tc_a(x, W1, degp)
    p1 = _agg(g1)
    g2 = _tc_b(p1, g1, degp, b1.reshape(1, FEAT), W2)
    p2 = _agg(g2)
    out = _tc_c(p2, g2, degp, b2.reshape(1, FEAT), Wfc, bfc.reshape(1, FEAT))
    return out


# speedup vs baseline: 14.7481x; 1.2238x over previous
import functools

import jax
import jax.numpy as jnp
from jax import lax
from jax.experimental import pallas as pl
from jax.experimental.pallas import tpu as pltpu
from jax.experimental.pallas import tpu_sc as plsc

N_NODES = 10000
N_EDGES = 320000
FEAT = 128
NC = 2
NS = 16
NW = NC * NS
EPW = N_EDGES // NW
K = 80
NCH = EPW // K
RPT = N_NODES // NS
DEGW = 16


def _sc_mesh():
    return plsc.VectorSubcoreMesh(
        core_axis_name="c", subcore_axis_name="s", num_cores=NC, num_subcores=NS
    )


def _deg_body(dst_hbm, zero_hbm, ones_hbm, out_hbm, acc, idx_d, ones):
    c = lax.axis_index("c")
    s = lax.axis_index("s")
    wid = c * NS + s

    pltpu.sync_copy(ones_hbm, ones)
    pltpu.sync_copy(zero_hbm, acc.at[pl.ds(s * RPT, RPT)])
    plsc.subcore_barrier()

    base = wid * EPW

    def body(i, carry):
        off = base + i * K
        pltpu.sync_copy(dst_hbm.at[pl.ds(off, K)], idx_d.at[0])
        pltpu.sync_copy(ones, acc.at[idx_d.at[0]], add=True)
        return carry

    lax.fori_loop(0, NCH, body, 0)
    plsc.subcore_barrier()
    pltpu.sync_copy(acc.at[pl.ds(s * RPT, RPT)], out_hbm.at[c, s])


def _agg_body(
    g_hbm, src_hbm, dst_hbm, zero_hbm, out_hbm,
    acc, sidx0, sidx1, didx0, didx1, rows0, rows1,
    sem_s0, sem_s1, sem_d0, sem_d1, sem_a, sem_b,
):
    c = lax.axis_index("c")
    s = lax.axis_index("s")
    wid = c * NS + s
    base = wid * EPW

    def load_idx(i, b, which):
        off = base + i * K
        if which == 0:
            buf, sem = (sidx0, sem_s0) if b == 0 else (sidx1, sem_s1)
            return pltpu.async_copy(src_hbm.at[pl.ds(off, K)], buf.at[0], sem)
        else:
            buf, sem = (didx0, sem_d0) if b == 0 else (didx1, sem_d1)
            return pltpu.async_copy(dst_hbm.at[pl.ds(off, K)], buf.at[0], sem)

    def scatter(b):
        if b == 0:
            pltpu.sync_copy(rows0, acc.at[didx0.at[0]], add=True)
        else:
            pltpu.sync_copy(rows1, acc.at[didx1.at[0]], add=True)

    pltpu.sync_copy(zero_hbm, acc.at[pl.ds(s * RPT, RPT)])
    plsc.subcore_barrier()

    def pair(j, carry):
        i0 = 2 * j
        d_s0 = load_idx(i0, 0, 0)
        d_d0 = load_idx(i0, 0, 1)
        d_s1 = load_idx(i0 + 1, 1, 0)
        d_d1 = load_idx(i0 + 1, 1, 1)
        d_s0.wait()
        g0 = pltpu.async_copy(g_hbm.at[sidx0.at[0]], rows0, sem_a)
        g0.wait()
        d_d0.wait()
        scatter(0)
        d_s1.wait()
        g1 = pltpu.async_copy(g_hbm.at[sidx1.at[0]], rows1, sem_b)
        g1.wait()
        d_d1.wait()
        scatter(1)
        return carry

    lax.fori_loop(0, NCH // 2, pair, 0)

    i_t = NCH - 1
    load_idx(i_t, 0, 0).wait()
    d_dt = load_idx(i_t, 0, 1)
    pltpu.async_copy(g_hbm.at[sidx0.at[0]], rows0, sem_a).wait()
    d_dt.wait()
    scatter(0)

    plsc.subcore_barrier()
    pltpu.sync_copy(acc.at[pl.ds(s * RPT, RPT)], out_hbm.at[c, s])


def _make_deg_kernel(interpret=False):
    return functools.partial(
        pl.kernel,
        out_type=jax.ShapeDtypeStruct((NC, NS, RPT, DEGW), jnp.float32),
        mesh=_sc_mesh(),
        scratch_types=[
            pltpu.VMEM_SHARED((N_NODES, DEGW), jnp.float32),
            pltpu.VMEM((1, K), jnp.int32),
            pltpu.VMEM((K, DEGW), jnp.float32),
        ],
        interpret=interpret,
    )(_deg_body)


def _make_agg_kernel(interpret=False):
    return functools.partial(
        pl.kernel,
        out_type=jax.ShapeDtypeStruct((NC, NS, RPT, FEAT), jnp.float32),
        mesh=_sc_mesh(),
        scratch_types=[
            pltpu.VMEM_SHARED((N_NODES, FEAT), jnp.float32),
            pltpu.VMEM((1, K), jnp.int32),
            pltpu.VMEM((1, K), jnp.int32),
            pltpu.VMEM((1, K), jnp.int32),
            pltpu.VMEM((1, K), jnp.int32),
            pltpu.VMEM((K, FEAT), jnp.float32),
            pltpu.VMEM((K, FEAT), jnp.float32),
        ] + [pltpu.SemaphoreType.DMA] * 6,
        interpret=interpret,
    )(_agg_body)


_deg_kernel = _make_deg_kernel()
_agg_kernel = _make_agg_kernel()


BM = 400


def _dinv_from(degp):
    deg = degp[0, :, 0] + degp[1, :, 0] + 1.0
    return lax.rsqrt(deg)


def _tc_a_body(x_ref, w_ref, degp_ref, o_ref):
    dinv = _dinv_from(degp_ref[...])
    h = jnp.dot(x_ref[...], w_ref[...], preferred_element_type=jnp.float32)
    o_ref[...] = h * dinv[:, None]


def _tc_b_body(p_ref, g_ref, degp_ref, b_ref, w_ref, o_ref):
    dinv = _dinv_from(degp_ref[...])
    p = p_ref[...]
    t = (p[0] + p[1] + g_ref[...]) * dinv[:, None] + b_ref[...]
    t = jnp.maximum(t, 0.0)
    o_ref[...] = jnp.dot(t, w_ref[...], preferred_element_type=jnp.float32) * dinv[:, None]


def _tc_c_body(p_ref, g_ref, degp_ref, b_ref, w_ref, bfc_ref, o_ref):
    dinv = _dinv_from(degp_ref[...])
    p = p_ref[...]
    t = (p[0] + p[1] + g_ref[...]) * dinv[:, None] + b_ref[...]
    t = jnp.maximum(t, 0.0)
    o_ref[...] = jnp.dot(t, w_ref[...], preferred_element_type=jnp.float32) + bfc_ref[...]


_BS_ROW = pl.BlockSpec((BM, FEAT), lambda i: (i, 0))
_BS_W = pl.BlockSpec((FEAT, FEAT), lambda i: (0, 0))
_BS_DEGP = pl.BlockSpec((NC, BM, DEGW), lambda i: (0, i, 0))
_BS_PART = pl.BlockSpec((NC, BM, FEAT), lambda i: (0, i, 0))
_BS_BIAS = pl.BlockSpec((1, FEAT), lambda i: (0, 0))
_OUT_SHAPE = jax.ShapeDtypeStruct((N_NODES, FEAT), jnp.float32)


def _tc_a(x, w1, degp):
    return pl.pallas_call(
        _tc_a_body,
        grid=(N_NODES // BM,),
        in_specs=[_BS_ROW, _BS_W, _BS_DEGP],
        out_specs=_BS_ROW,
        out_shape=_OUT_SHAPE,
    )(x, w1, degp)


def _tc_b(p, g, degp, b, w):
    return pl.pallas_call(
        _tc_b_body,
        grid=(N_NODES // BM,),
        in_specs=[_BS_PART, _BS_ROW, _BS_DEGP, _BS_BIAS, _BS_W],
        out_specs=_BS_ROW,
        out_shape=_OUT_SHAPE,
    )(p, g, degp, b, w)


def _tc_c(p, g, degp, b, w, bfc):
    return pl.pallas_call(
        _tc_c_body,
        grid=(N_NODES // BM,),
        in_specs=[_BS_PART, _BS_ROW, _BS_DEGP, _BS_BIAS, _BS_W, _BS_BIAS],
        out_specs=_BS_ROW,
        out_shape=_OUT_SHAPE,
    )(p, g, degp, b, w, bfc)


def kernel(x, edge_index, W1, b1, W2, b2, Wfc, bfc):
    src = edge_index[0]
    dst = edge_index[1]
    zero_f = jnp.zeros((RPT, FEAT), jnp.float32)
    zero_d = jnp.zeros((RPT, DEGW), jnp.float32)

    degp = _deg_kernel(
        dst, zero_d, jnp.ones((K, DEGW), jnp.float32)
    ).reshape(NC, N_NODES, DEGW)

    def _agg(g):
        return _agg_kernel(g, src, dst, zero_f).reshape(NC, N_NODES, FEAT)
    g1 = _tc_a(x, W1, degp)
    p1 = _agg(g1)
    g2 = _tc_b(p1, g1, degp, b1.reshape(1, FEAT), W2)
    p2 = _agg(g2)
    out = _tc_c(p2, g2, degp, b2.reshape(1, FEAT), Wfc, bfc.reshape(1, FEAT))
    return out


# speedup vs baseline: 15.3109x; 1.0382x over previous
import functools

import jax
import jax.numpy as jnp
from jax import lax
from jax.experimental import pallas as pl
from jax.experimental.pallas import tpu as pltpu
from jax.experimental.pallas import tpu_sc as plsc

N_NODES = 10000
N_EDGES = 320000
FEAT = 128
NC = 2
NS = 16
NW = NC * NS
EPW = N_EDGES // NW
K = 80
NCH = EPW // K
RPT = N_NODES // NS
DEGW = 16


def _sc_mesh():
    return plsc.VectorSubcoreMesh(
        core_axis_name="c", subcore_axis_name="s", num_cores=NC, num_subcores=NS
    )


def _deg_body(dst_hbm, zero_hbm, ones_hbm, out_hbm, acc, idx0, idx1, ones, sem0, sem1):
    c = lax.axis_index("c")
    s = lax.axis_index("s")
    wid = c * NS + s

    pltpu.sync_copy(ones_hbm, ones)
    pltpu.sync_copy(zero_hbm, acc.at[pl.ds(s * RPT, RPT)])
    plsc.subcore_barrier()

    base = wid * EPW

    def pair(j, carry):
        i0 = 2 * j
        d0 = pltpu.async_copy(dst_hbm.at[pl.ds(base + i0 * K, K)], idx0.at[0], sem0)
        d1 = pltpu.async_copy(dst_hbm.at[pl.ds(base + (i0 + 1) * K, K)], idx1.at[0], sem1)
        d0.wait()
        pltpu.sync_copy(ones, acc.at[idx0.at[0]], add=True)
        d1.wait()
        pltpu.sync_copy(ones, acc.at[idx1.at[0]], add=True)
        return carry

    lax.fori_loop(0, NCH // 2, pair, 0)
    pltpu.async_copy(dst_hbm.at[pl.ds(base + (NCH - 1) * K, K)], idx0.at[0], sem0).wait()
    pltpu.sync_copy(ones, acc.at[idx0.at[0]], add=True)
    plsc.subcore_barrier()
    pltpu.sync_copy(acc.at[pl.ds(s * RPT, RPT)], out_hbm.at[c, s])


def _agg_body(
    g_hbm, src_hbm, dst_hbm, zero_hbm, out_hbm,
    acc, sidx0, sidx1, didx0, didx1, rows0, rows1,
    sem_s0, sem_s1, sem_d0, sem_d1, sem_a, sem_b,
):
    c = lax.axis_index("c")
    s = lax.axis_index("s")
    wid = c * NS + s
    base = wid * EPW

    def load_idx(i, b, which):
        off = base + i * K
        if which == 0:
            buf, sem = (sidx0, sem_s0) if b == 0 else (sidx1, sem_s1)
            return pltpu.async_copy(src_hbm.at[pl.ds(off, K)], buf.at[0], sem)
        else:
            buf, sem = (didx0, sem_d0) if b == 0 else (didx1, sem_d1)
            return pltpu.async_copy(dst_hbm.at[pl.ds(off, K)], buf.at[0], sem)

    def scatter(b):
        if b == 0:
            pltpu.sync_copy(rows0, acc.at[didx0.at[0]], add=True)
        else:
            pltpu.sync_copy(rows1, acc.at[didx1.at[0]], add=True)

    pltpu.sync_copy(zero_hbm, acc.at[pl.ds(s * RPT, RPT)])
    plsc.subcore_barrier()

    def pair(j, carry):
        i0 = 2 * j
        d_s0 = load_idx(i0, 0, 0)
        d_d0 = load_idx(i0, 0, 1)
        d_s1 = load_idx(i0 + 1, 1, 0)
        d_d1 = load_idx(i0 + 1, 1, 1)
        d_s0.wait()
        g0 = pltpu.async_copy(g_hbm.at[sidx0.at[0]], rows0, sem_a)
        g0.wait()
        d_d0.wait()
        scatter(0)
        d_s1.wait()
        g1 = pltpu.async_copy(g_hbm.at[sidx1.at[0]], rows1, sem_b)
        g1.wait()
        d_d1.wait()
        scatter(1)
        return carry

    lax.fori_loop(0, NCH // 2, pair, 0)

    i_t = NCH - 1
    load_idx(i_t, 0, 0).wait()
    d_dt = load_idx(i_t, 0, 1)
    pltpu.async_copy(g_hbm.at[sidx0.at[0]], rows0, sem_a).wait()
    d_dt.wait()
    scatter(0)

    plsc.subcore_barrier()
    pltpu.sync_copy(acc.at[pl.ds(s * RPT, RPT)], out_hbm.at[c, s])


def _make_deg_kernel(interpret=False):
    return functools.partial(
        pl.kernel,
        out_type=jax.ShapeDtypeStruct((NC, NS, RPT, DEGW), jnp.float32),
        mesh=_sc_mesh(),
        scratch_types=[
            pltpu.VMEM_SHARED((N_NODES, DEGW), jnp.float32),
            pltpu.VMEM((1, K), jnp.int32),
            pltpu.VMEM((1, K), jnp.int32),
            pltpu.VMEM((K, DEGW), jnp.float32),
            pltpu.SemaphoreType.DMA,
            pltpu.SemaphoreType.DMA,
        ],
        interpret=interpret,
    )(_deg_body)


def _make_agg_kernel(interpret=False):
    return functools.partial(
        pl.kernel,
        out_type=jax.ShapeDtypeStruct((NC, NS, RPT, FEAT), jnp.float32),
        mesh=_sc_mesh(),
        scratch_types=[
            pltpu.VMEM_SHARED((N_NODES, FEAT), jnp.float32),
            pltpu.VMEM((1, K), jnp.int32),
            pltpu.VMEM((1, K), jnp.int32),
            pltpu.VMEM((1, K), jnp.int32),
            pltpu.VMEM((1, K), jnp.int32),
            pltpu.VMEM((K, FEAT), jnp.float32),
            pltpu.VMEM((K, FEAT), jnp.float32),
        ] + [pltpu.SemaphoreType.DMA] * 6,
        interpret=interpret,
    )(_agg_body)


_deg_kernel = _make_deg_kernel()
_agg_kernel = _make_agg_kernel()


BM = 400


def _dinv_from(degp):
    deg = degp[0, :, 0] + degp[1, :, 0] + 1.0
    return lax.rsqrt(deg)


def _tc_a_body(x_ref, w_ref, degp_ref, o_ref):
    dinv = _dinv_from(degp_ref[...])
    h = jnp.dot(x_ref[...], w_ref[...], preferred_element_type=jnp.float32)
    o_ref[...] = h * dinv[:, None]


def _tc_b_body(p_ref, g_ref, degp_ref, b_ref, w_ref, o_ref):
    dinv = _dinv_from(degp_ref[...])
    p = p_ref[...]
    t = (p[0] + p[1] + g_ref[...]) * dinv[:, None] + b_ref[...]
    t = jnp.maximum(t, 0.0)
    o_ref[...] = jnp.dot(t, w_ref[...], preferred_element_type=jnp.float32) * dinv[:, None]


def _tc_c_body(p_ref, g_ref, degp_ref, b_ref, w_ref, bfc_ref, o_ref):
    dinv = _dinv_from(degp_ref[...])
    p = p_ref[...]
    t = (p[0] + p[1] + g_ref[...]) * dinv[:, None] + b_ref[...]
    t = jnp.maximum(t, 0.0)
    o_ref[...] = jnp.dot(t, w_ref[...], preferred_element_type=jnp.float32) + bfc_ref[...]


_BS_ROW = pl.BlockSpec((BM, FEAT), lambda i: (i, 0))
_BS_W = pl.BlockSpec((FEAT, FEAT), lambda i: (0, 0))
_BS_DEGP = pl.BlockSpec((NC, BM, DEGW), lambda i: (0, i, 0))
_BS_PART = pl.BlockSpec((NC, BM, FEAT), lambda i: (0, i, 0))
_BS_BIAS = pl.BlockSpec((1, FEAT), lambda i: (0, 0))
_OUT_SHAPE = jax.ShapeDtypeStruct((N_NODES, FEAT), jnp.float32)


def _tc_a(x, w1, degp):
    return pl.pallas_call(
        _tc_a_body,
        grid=(N_NODES // BM,),
        in_specs=[_BS_ROW, _BS_W, _BS_DEGP],
        out_specs=_BS_ROW,
        out_shape=_OUT_SHAPE,
    )(x, w1, degp)


def _tc_b(p, g, degp, b, w):
    return pl.pallas_call(
        _tc_b_body,
        grid=(N_NODES // BM,),
        in_specs=[_BS_PART, _BS_ROW, _BS_DEGP, _BS_BIAS, _BS_W],
        out_specs=_BS_ROW,
        out_shape=_OUT_SHAPE,
    )(p, g, degp, b, w)


def _tc_c(p, g, degp, b, w, bfc):
    return pl.pallas_call(
        _tc_c_body,
        grid=(N_NODES // BM,),
        in_specs=[_BS_PART, _BS_ROW, _BS_DEGP, _BS_BIAS, _BS_W, _BS_BIAS],
        out_specs=_BS_ROW,
        out_shape=_OUT_SHAPE,
    )(p, g, degp, b, w, bfc)


def kernel(x, edge_index, W1, b1, W2, b2, Wfc, bfc):
    src = edge_index[0]
    dst = edge_index[1]
    zero_f = jnp.zeros((RPT, FEAT), jnp.float32)
    zero_d = jnp.zeros((RPT, DEGW), jnp.float32)

    degp = _deg_kernel(
        dst, zero_d, jnp.ones((K, DEGW), jnp.float32)
    ).reshape(NC, N_NODES, DEGW)

    def _agg(g):
        return _agg_kernel(g, src, dst, zero_f).reshape(NC, N_NODES, FEAT)
    g1 = _tc_a(x, W1, degp)
    p1 = _agg(g1)
    g2 = _tc_b(p1, g1, degp, b1.reshape(1, FEAT), W2)
    p2 = _agg(g2)
    out = _tc_c(p2, g2, degp, b2.reshape(1, FEAT), Wfc, bfc.reshape(1, FEAT))
    return out
